# baseline (speedup 1.0000x reference)
import functools

import jax
import jax.numpy as jnp
from jax import lax
from jax.experimental import pallas as pl
from jax.experimental.pallas import tpu as pltpu
from jax.experimental.pallas import tpu_sc as plsc

H = 16
NB = 33
S = 2048
L = 16
NC = 2
NS = 16
NW = NC * NS
ROWS_PER_W = H * S // NW
UREV = 2 * S
BLK = 8
NBLK = ROWS_PER_W // BLK


def _rpb_body(table_hbm, out_hbm, table_v, u0, u1, u2, u3, u4, u5, u6, u7,
              stg0, stg1, stg2, stg3, sem0, sem1, sem2, sem3):
    urevs = (u0, u1, u2, u3, u4, u5, u6, u7)
    stgs = (stg0, stg1, stg2, stg3)
    sems = (sem0, sem1, sem2, sem3)
    cid = lax.axis_index("c")
    sid = lax.axis_index("s")
    wid = sid * NC + cid
    h = wid // 2
    half = wid % 2
    row0 = half * ROWS_PER_W

    pltpu.sync_copy(table_hbm.at[pl.ds(h * NB * L, NB * L)], table_v)
    wv = [table_v[pl.ds(b * L, L)] for b in range(NB)]

    def fill_lo(k, carry):
        for s in range(8):
            urevs[s][pl.ds(k * L, L)] = wv[NB - 1]
        return carry

    lax.fori_loop(0, 125, fill_lo, 0)

    def fill_hi(k, carry):
        for s in range(8):
            urevs[s][pl.ds(k * L, L)] = wv[0]
        return carry

    lax.fori_loop(128, UREV // L, fill_hi, 0)

    lane = lax.iota(jnp.int32, L)
    for k in (125, 126, 127):
        for s in range(8):
            t = k * L + lane
            d = jnp.clip((S - 1) - s - t, 0, NB - 1)
            v = wv[NB - 1]
            for b in range(NB - 1):
                v = jnp.where(d == b, wv[b], v)
            urevs[s][pl.ds(k * L, L)] = v

    s_of = [(7 - r8) % 8 for r8 in range(BLK)]

    def paint_chunk(par, r8, i_t, col):
        stgs[par][r8, pl.ds(col, L)] = urevs[s_of[r8]][
            pl.ds(2040 - i_t + col, L)
        ]

    def paint_full(par, i_t):
        def body(c, carry):
            for r8 in range(BLK):
                paint_chunk(par, r8, i_t, c * L)
            return carry

        lax.fori_loop(0, S // L, body, 0)

    def fire(par, i_t):
        pltpu.async_copy(
            stgs[par], out_hbm.at[0, h, pl.ds(i_t, BLK), :], sems[par]
        )

    def wait(par):
        pltpu.make_async_copy(
            stgs[par], out_hbm.at[0, h, pl.ds(row0, BLK), :], sems[par]
        ).wait()

    for par in range(4):
        paint_full(par, row0 + BLK * par)
        fire(par, row0 + BLK * par)

    def quad(bq, carry):
        for par in range(4):
            b = 4 * bq + par
            i_t = row0 + BLK * b
            wait(par)
            for r8 in range(BLK):
                i_r = i_t + r8
                c0 = jnp.clip((i_r - 64) // L, 0, (S - 96) // L)
                for cc in range(6):
                    paint_chunk(par, r8, i_t, (c0 + cc) * L)
            fire(par, i_t)
        return carry

    lax.fori_loop(1, NBLK // 4, quad, 0)

    for par in range(4):
        wait(par)


@jax.jit
def _rpb(table):
    mesh = plsc.VectorSubcoreMesh(core_axis_name="c", subcore_axis_name="s")
    out = pl.kernel(
        _rpb_body,
        mesh=mesh,
        out_type=jax.ShapeDtypeStruct((1, H, S, S), jnp.float32),
        scratch_types=[
            pltpu.VMEM((NB * L,), jnp.float32),
            *[pltpu.VMEM((UREV,), jnp.float32) for _ in range(8)],
            *[pltpu.VMEM((BLK, S), jnp.float32) for _ in range(4)],
            *[pltpu.SemaphoreType.DMA for _ in range(4)],
        ],
    )(jnp.broadcast_to(table.T[:, :, None], (H, NB, L)).reshape(H * NB * L))
    return out


def kernel(seq_len, table):
    del seq_len
    return _rpb(table.astype(jnp.float32))


# speedup vs baseline: 1.0057x; 1.0057x over previous
import functools

import jax
import jax.numpy as jnp
from jax import lax
from jax.experimental import pallas as pl
from jax.experimental.pallas import tpu as pltpu
from jax.experimental.pallas import tpu_sc as plsc

H = 16
NB = 33
S = 2048
L = 16
NC = 2
NS = 16
NW = NC * NS
ROWS_PER_W = H * S // NW
UREV = 2 * S
BLK = 16
NBLK = ROWS_PER_W // BLK
PAINT_CH = (2 * BLK + 32) // L + 2


def _rpb_body(table_hbm, out_hbm, table_v, u0, u1, u2, u3, u4, u5, u6, u7,
              stg0, stg1, sem0, sem1):
    urevs = (u0, u1, u2, u3, u4, u5, u6, u7)
    stgs = (stg0, stg1)
    sems = (sem0, sem1)
    cid = lax.axis_index("c")
    sid = lax.axis_index("s")
    wid = sid * NC + cid
    h = wid // 2
    half = wid % 2
    row0 = half * ROWS_PER_W

    pltpu.sync_copy(table_hbm.at[pl.ds(h * NB * L, NB * L)], table_v)
    wv = [table_v[pl.ds(b * L, L)] for b in range(NB)]

    def fill_lo(k, carry):
        for s in range(8):
            urevs[s][pl.ds(k * L, L)] = wv[NB - 1]
        return carry

    lax.fori_loop(0, 125, fill_lo, 0)

    def fill_hi(k, carry):
        for s in range(8):
            urevs[s][pl.ds(k * L, L)] = wv[0]
        return carry

    lax.fori_loop(128, UREV // L, fill_hi, 0)

    lane = lax.iota(jnp.int32, L)
    for k in (125, 126, 127):
        for s in range(8):
            t = k * L + lane
            d = jnp.clip((S - 1) - s - t, 0, NB - 1)
            v = wv[NB - 1]
            for b in range(NB - 1):
                v = jnp.where(d == b, wv[b], v)
            urevs[s][pl.ds(k * L, L)] = v

    s_of = [(7 - r8) % 8 for r8 in range(BLK)]

    def paint_chunk(par, r8, i_t, col):
        stgs[par][r8, pl.ds(col, L)] = urevs[s_of[r8]][
            pl.ds(2040 - i_t + col, L)
        ]

    def paint_full(par, i_t):
        def body(c, carry):
            for r8 in range(BLK):
                paint_chunk(par, r8, i_t, c * L)
            return carry

        lax.fori_loop(0, S // L, body, 0)

    def fire(par, i_t):
        pltpu.async_copy(
            stgs[par], out_hbm.at[0, h, pl.ds(i_t, BLK), :], sems[par]
        )

    def wait(par):
        pltpu.make_async_copy(
            stgs[par], out_hbm.at[0, h, pl.ds(row0, BLK), :], sems[par]
        ).wait()

    paint_full(0, row0)
    fire(0, row0)
    paint_full(1, row0 + BLK)
    fire(1, row0 + BLK)

    def pair(bp, carry):
        for par in range(2):
            b = 2 * bp + par
            i_t = row0 + BLK * b
            wait(par)
            for r8 in range(BLK):
                i_r = i_t + r8
                c0 = jnp.clip((i_r - (2 * BLK + 32)) // L, 0,
                              (S - PAINT_CH * L) // L)
                for cc in range(PAINT_CH):
                    paint_chunk(par, r8, i_t, (c0 + cc) * L)
            fire(par, i_t)
        return carry

    lax.fori_loop(1, NBLK // 2, pair, 0)

    wait(0)
    wait(1)


@jax.jit
def _rpb(table):
    mesh = plsc.VectorSubcoreMesh(core_axis_name="c", subcore_axis_name="s")
    out = pl.kernel(
        _rpb_body,
        mesh=mesh,
        out_type=jax.ShapeDtypeStruct((1, H, S, S), jnp.float32),
        scratch_types=[
            pltpu.VMEM((NB * L,), jnp.float32),
            *[pltpu.VMEM((UREV,), jnp.float32) for _ in range(8)],
            pltpu.VMEM((BLK, S), jnp.float32),
            pltpu.VMEM((BLK, S), jnp.float32),
            pltpu.SemaphoreType.DMA,
            pltpu.SemaphoreType.DMA,
        ],
    )(jnp.broadcast_to(table.T[:, :, None], (H, NB, L)).reshape(H * NB * L))
    return out


def kernel(seq_len, table):
    del seq_len
    return _rpb(table.astype(jnp.float32))


# speedup vs baseline: 1.0916x; 1.0854x over previous
import functools

import jax
import jax.numpy as jnp
from jax import lax
from jax.experimental import pallas as pl
from jax.experimental.pallas import tpu as pltpu
from jax.experimental.pallas import tpu_sc as plsc

H = 16
NB = 33
S = 2048
L = 16
NC = 2
NS = 16
NW = NC * NS
ROWS_PER_W = H * S // NW
UREV = 2 * S
BLK = 8
NBLK = ROWS_PER_W // BLK
PAINT_CH = (2 * BLK + 32) // L + 2


def _rpb_body(table_hbm, out_hbm, table_v, u0, u1, u2, u3, u4, u5, u6, u7,
              stg0, stg1, sem0, sem1):
    urevs = (u0, u1, u2, u3, u4, u5, u6, u7)
    stgs = (stg0, stg1)
    sems = (sem0, sem1)
    cid = lax.axis_index("c")
    sid = lax.axis_index("s")
    wid = sid * NC + cid
    h = wid // 2
    half = wid % 2
    row0 = half * ROWS_PER_W

    pltpu.sync_copy(table_hbm.at[pl.ds(h * NB * L, NB * L)], table_v)
    wv = [table_v[pl.ds(b * L, L)] for b in range(NB)]

    def fill_lo(k, carry):
        for s in range(8):
            urevs[s][pl.ds(k * L, L)] = wv[NB - 1]
        return carry

    lax.fori_loop(0, 125, fill_lo, 0)

    def fill_hi(k, carry):
        for s in range(8):
            urevs[s][pl.ds(k * L, L)] = wv[0]
        return carry

    lax.fori_loop(128, UREV // L, fill_hi, 0)

    lane = lax.iota(jnp.int32, L)
    for k in (125, 126, 127):
        for s in range(8):
            t = k * L + lane
            d = jnp.clip((S - 1) - s - t, 0, NB - 1)
            v = wv[NB - 1]
            for b in range(NB - 1):
                v = jnp.where(d == b, wv[b], v)
            urevs[s][pl.ds(k * L, L)] = v

    s_of = [(7 - r8) % 8 for r8 in range(BLK)]

    def paint_chunk(par, r8, i_t, col):
        stgs[par][r8, pl.ds(col, L)] = urevs[s_of[r8]][
            pl.ds(2040 - i_t + col, L)
        ]

    def paint_full(par, i_t):
        def body(c, carry):
            for r8 in range(BLK):
                paint_chunk(par, r8, i_t, c * L)
            return carry

        lax.fori_loop(0, S // L, body, 0)

    def fire(par, i_t):
        pltpu.async_copy(
            stgs[par], out_hbm.at[0, h, pl.ds(i_t, BLK), :], sems[par]
        )

    def wait(par):
        pltpu.make_async_copy(
            stgs[par], out_hbm.at[0, h, pl.ds(row0, BLK), :], sems[par]
        ).wait()

    paint_full(0, row0)
    fire(0, row0)
    paint_full(1, row0 + BLK)
    fire(1, row0 + BLK)

    def pair(bp, carry):
        for par in range(2):
            b = 2 * bp + par
            i_t = row0 + BLK * b
            wait(par)
            for r8 in range(BLK):
                i_r = i_t + r8
                c0 = jnp.clip((i_r - (2 * BLK + 32)) // L, 0,
                              (S - PAINT_CH * L) // L)
                for cc in range(PAINT_CH):
                    paint_chunk(par, r8, i_t, (c0 + cc) * L)
            fire(par, i_t)
        return carry

    lax.fori_loop(1, NBLK // 2, pair, 0)

    wait(0)
    wait(1)


@jax.jit
def _rpb(table):
    mesh = plsc.VectorSubcoreMesh(core_axis_name="c", subcore_axis_name="s")
    out = pl.kernel(
        _rpb_body,
        mesh=mesh,
        out_type=jax.ShapeDtypeStruct((1, H, S, S), jnp.float32),
        scratch_types=[
            pltpu.VMEM((NB * L,), jnp.float32),
            *[pltpu.VMEM((UREV,), jnp.float32) for _ in range(8)],
            pltpu.VMEM((BLK, S), jnp.float32),
            pltpu.VMEM((BLK, S), jnp.float32),
            pltpu.SemaphoreType.DMA,
            pltpu.SemaphoreType.DMA,
        ],
    )(jnp.broadcast_to(table.T[:, :, None], (H, NB, L)).reshape(H * NB * L))
    return out


def kernel(seq_len, table):
    del seq_len
    return _rpb(table.astype(jnp.float32))
